# baseline (speedup 1.0000x reference)
import functools

import jax
import jax.numpy as jnp
from jax import lax
from jax.experimental import pallas as pl
from jax.experimental.pallas import tpu as pltpu
from jax.experimental.pallas import tpu_sc as plsc

B = 64
C = 64
HW = 1024
K = 512
N = B * HW
NELEM = N * C


def _tc_body(z_ref, emb_ref, idx_ref, loss_ref, perp_ref,
             acc, mean_s, std_s, err_s, cnt):
    i = pl.program_id(0)

    @pl.when(i == 0)
    def _init():
        acc[...] = jnp.zeros_like(acc)

    @pl.when(i < B)
    def _phase_sum():
        acc[...] += z_ref[0]

    @pl.when(i == B)
    def _fin_mean():
        mean_s[...] = jnp.sum(acc[...], axis=1, keepdims=True) / N
        acc[...] = jnp.zeros_like(acc)

    @pl.when(jnp.logical_and(i >= B, i < 2 * B))
    def _phase_sq():
        d = z_ref[0] - mean_s[...]
        acc[...] += d * d

    @pl.when(i == 2 * B)
    def _fin_std():
        var = jnp.sum(acc[...], axis=1, keepdims=True) / (N - 1)
        std_s[...] = jnp.maximum(jnp.sqrt(var), 1e-5)
        err_s[0, 0] = 0.0
        cnt[...] = jnp.zeros_like(cnt)

    @pl.when(i >= 2 * B)
    def _phase_main():
        zn = z_ref[0] / std_s[...]
        emb = emb_ref[...]
        mm = lax.dot_general(emb, zn, (((1,), (0,)), ((), ())),
                             preferred_element_type=jnp.float32)
        esq = jnp.sum(emb * emb, axis=1, keepdims=True)
        zsq = jnp.sum(zn * zn, axis=0, keepdims=True)
        dist = (esq - 2.0 * mm) + zsq
        md = jnp.min(dist, axis=0)
        kio = lax.broadcasted_iota(jnp.int32, (K, HW), 0)
        idx = jnp.min(jnp.where(dist == md[None, :], kio, K), axis=0)
        err_s[0, 0] += jnp.sum(md)
        oh = (kio == idx[None, :]).astype(jnp.float32)
        cnt[...] += jnp.sum(oh, axis=1, keepdims=True)
        idx_ref[0, 0, :] = idx

    @pl.when(i == 3 * B - 1)
    def _finalize():
        loss_ref[0, 0] = 1.25 * err_s[0, 0] / NELEM
        p = cnt[...] / N
        plogp = p * jnp.log(jnp.maximum(p, 1e-10))
        perp_ref[0, 0] = jnp.exp(-jnp.sum(plogp))


def _sc_body(emb_hbm, idx_hbm, out_hbm, emb_v, idx_v, obuf):
    cid = lax.axis_index("c")
    sid = lax.axis_index("s")
    wid = sid * 2 + cid
    pltpu.sync_copy(emb_hbm, emb_v)

    def per_batch(bl, carry):
        b = wid * 2 + bl
        pltpu.sync_copy(idx_hbm.at[b], idx_v)

        def per_group(g, carry2):
            base = idx_v[pl.ds(g * 16, 16)] * C

            def per_chan(c, carry3):
                vals = plsc.load_gather(emb_v, [base + c])
                obuf[c, pl.ds(g * 16, 16)] = vals
                return carry3

            return lax.fori_loop(0, C, per_chan, carry2)

        lax.fori_loop(0, HW // 16, per_group, carry)
        pltpu.sync_copy(obuf, out_hbm.at[b])
        return carry

    lax.fori_loop(0, 2, per_batch, 0)


def _make_sc_gather():
    mesh = plsc.VectorSubcoreMesh(core_axis_name="c", subcore_axis_name="s")
    return functools.partial(
        pl.kernel,
        mesh=mesh,
        compiler_params=pltpu.CompilerParams(needs_layout_passes=False),
        out_type=jax.ShapeDtypeStruct((B, C, HW), jnp.float32),
        scratch_types=[
            pltpu.VMEM((K * C,), jnp.float32),
            pltpu.VMEM((HW,), jnp.int32),
            pltpu.VMEM((C, HW), jnp.float32),
        ],
    )(_sc_body)


def kernel(z_e, emb_w):
    z3 = z_e.reshape(B, C, HW)
    idx3, loss, perp = pl.pallas_call(
        _tc_body,
        grid=(3 * B,),
        in_specs=[
            pl.BlockSpec((1, C, HW), lambda i: (i % B, 0, 0)),
            pl.BlockSpec((K, C), lambda i: (0, 0)),
        ],
        out_specs=[
            pl.BlockSpec((1, 1, HW), lambda i: (jnp.maximum(i - 2 * B, 0), 0, 0)),
            pl.BlockSpec(memory_space=pltpu.SMEM),
            pl.BlockSpec(memory_space=pltpu.SMEM),
        ],
        out_shape=[
            jax.ShapeDtypeStruct((B, 1, HW), jnp.int32),
            jax.ShapeDtypeStruct((1, 1), jnp.float32),
            jax.ShapeDtypeStruct((1, 1), jnp.float32),
        ],
        scratch_shapes=[
            pltpu.VMEM((C, HW), jnp.float32),
            pltpu.VMEM((C, 1), jnp.float32),
            pltpu.VMEM((C, 1), jnp.float32),
            pltpu.SMEM((1, 1), jnp.float32),
            pltpu.VMEM((K, 1), jnp.float32),
        ],
    )(z3, emb_w)
    idx2 = idx3.reshape(B, HW)
    zq = _make_sc_gather()(emb_w.reshape(-1), idx2)
    z_q_st = zq.reshape(z_e.shape)
    indices = idx2.reshape(B, 32, 32)
    return (z_q_st, loss[0, 0], perp[0, 0], indices)


# speedup vs baseline: 1.1436x; 1.1436x over previous
import functools

import jax
import jax.numpy as jnp
from jax import lax
from jax.experimental import pallas as pl
from jax.experimental.pallas import tpu as pltpu
from jax.experimental.pallas import tpu_sc as plsc

B = 64
C = 64
HW = 1024
K = 512
N = B * HW
NELEM = N * C


def _tc_body(z_ref, emb_ref, idx_ref, loss_ref, perp_ref,
             acc, mean_s, std_s, err_s, cnt):
    i = pl.program_id(0)

    @pl.when(i == 0)
    def _init():
        acc[...] = jnp.zeros_like(acc)

    @pl.when(i < B)
    def _phase_sum():
        acc[...] += z_ref[0]

    @pl.when(i == B)
    def _fin_mean():
        mean_s[...] = jnp.sum(acc[...], axis=1, keepdims=True) / N
        acc[...] = jnp.zeros_like(acc)

    @pl.when(jnp.logical_and(i >= B, i < 2 * B))
    def _phase_sq():
        d = z_ref[0] - mean_s[...]
        acc[...] += d * d

    @pl.when(i == 2 * B)
    def _fin_std():
        var = jnp.sum(acc[...], axis=1, keepdims=True) / (N - 1)
        std_s[...] = jnp.maximum(jnp.sqrt(var), 1e-5)
        err_s[0, 0] = 0.0
        cnt[...] = jnp.zeros_like(cnt)

    @pl.when(i >= 2 * B)
    def _phase_main():
        zn = z_ref[0] / std_s[...]
        emb = emb_ref[...]
        mm = lax.dot_general(emb, zn, (((1,), (0,)), ((), ())),
                             preferred_element_type=jnp.float32)
        esq = jnp.sum(emb * emb, axis=1, keepdims=True)
        zsq = jnp.sum(zn * zn, axis=0, keepdims=True)
        dist = (esq - 2.0 * mm) + zsq
        md = jnp.min(dist, axis=0)
        kio = lax.broadcasted_iota(jnp.int32, (K, HW), 0)
        idx = jnp.min(jnp.where(dist == md[None, :], kio, K), axis=0)
        err_s[0, 0] += jnp.sum(md)
        oh = (kio == idx[None, :]).astype(jnp.float32)
        cnt[...] += jnp.sum(oh, axis=1, keepdims=True)
        idx_ref[0, 0, :] = idx

    @pl.when(i == 3 * B - 1)
    def _finalize():
        loss_ref[0, 0] = 1.25 * err_s[0, 0] / NELEM
        p = cnt[...] / N
        plogp = p * jnp.log(jnp.maximum(p, 1e-10))
        perp_ref[0, 0] = jnp.exp(-jnp.sum(plogp))


def _sc_body(emb_hbm, idx_hbm, out_hbm, emb_v, idx_v, obuf):
    cid = lax.axis_index("c")
    sid = lax.axis_index("s")
    wid = sid * 2 + cid
    pltpu.sync_copy(emb_hbm, emb_v)

    def per_batch(bl, carry):
        b = wid * 2 + bl
        pltpu.sync_copy(idx_hbm.at[b], idx_v)

        @plsc.parallel_loop(0, HW // 16, unroll=2)
        def per_group(g):
            base = idx_v[pl.ds(g * 16, 16)] * C
            for c in range(C):
                obuf[c, pl.ds(g * 16, 16)] = plsc.load_gather(emb_v, [base + c])

        pltpu.sync_copy(obuf, out_hbm.at[b])
        return carry

    lax.fori_loop(0, 2, per_batch, 0)


def _make_sc_gather():
    mesh = plsc.VectorSubcoreMesh(core_axis_name="c", subcore_axis_name="s")
    return functools.partial(
        pl.kernel,
        mesh=mesh,
        compiler_params=pltpu.CompilerParams(needs_layout_passes=False),
        out_type=jax.ShapeDtypeStruct((B, C, HW), jnp.float32),
        scratch_types=[
            pltpu.VMEM((K * C,), jnp.float32),
            pltpu.VMEM((HW,), jnp.int32),
            pltpu.VMEM((C, HW), jnp.float32),
        ],
    )(_sc_body)


def kernel(z_e, emb_w):
    z3 = z_e.reshape(B, C, HW)
    idx3, loss, perp = pl.pallas_call(
        _tc_body,
        grid=(3 * B,),
        in_specs=[
            pl.BlockSpec((1, C, HW), lambda i: (i % B, 0, 0)),
            pl.BlockSpec((K, C), lambda i: (0, 0)),
        ],
        out_specs=[
            pl.BlockSpec((1, 1, HW), lambda i: (jnp.maximum(i - 2 * B, 0), 0, 0)),
            pl.BlockSpec(memory_space=pltpu.SMEM),
            pl.BlockSpec(memory_space=pltpu.SMEM),
        ],
        out_shape=[
            jax.ShapeDtypeStruct((B, 1, HW), jnp.int32),
            jax.ShapeDtypeStruct((1, 1), jnp.float32),
            jax.ShapeDtypeStruct((1, 1), jnp.float32),
        ],
        scratch_shapes=[
            pltpu.VMEM((C, HW), jnp.float32),
            pltpu.VMEM((C, 1), jnp.float32),
            pltpu.VMEM((C, 1), jnp.float32),
            pltpu.SMEM((1, 1), jnp.float32),
            pltpu.VMEM((K, 1), jnp.float32),
        ],
    )(z3, emb_w)
    idx2 = idx3.reshape(B, HW)
    zq = _make_sc_gather()(emb_w.reshape(-1), idx2)
    z_q_st = zq.reshape(z_e.shape)
    indices = idx2.reshape(B, 32, 32)
    return (z_q_st, loss[0, 0], perp[0, 0], indices)


# speedup vs baseline: 1.8463x; 1.6145x over previous
import functools

import jax
import jax.numpy as jnp
from jax import lax
from jax.experimental import pallas as pl
from jax.experimental.pallas import tpu as pltpu
from jax.experimental.pallas import tpu_sc as plsc

B = 64
C = 64
HW = 1024
K = 512
N = B * HW
NELEM = N * C
BB = 4
SA = B // BB


def _tc_body(z_ref, emb_ref, idx_ref, loss_ref, perp_ref, embt_ref,
             acc, mean_s, std_s, err_s, cnt):
    i = pl.program_id(0)

    @pl.when(i == 0)
    def _init():
        acc[...] = jnp.zeros_like(acc)

    @pl.when(i < SA)
    def _phase_sum():
        for j in range(BB):
            acc[...] += z_ref[j]

    @pl.when(i == SA)
    def _fin_mean():
        mean_s[...] = jnp.sum(acc[...], axis=1, keepdims=True) / N
        acc[...] = jnp.zeros_like(acc)

    @pl.when(jnp.logical_and(i >= SA, i < 2 * SA))
    def _phase_sq():
        for j in range(BB):
            d = z_ref[j] - mean_s[...]
            acc[...] += d * d

    @pl.when(i == 2 * SA)
    def _fin_std():
        var = jnp.sum(acc[...], axis=1, keepdims=True) / (N - 1)
        std_s[...] = jnp.maximum(jnp.sqrt(var), 1e-5)
        embt_ref[...] = emb_ref[...].T
        err_s[0, 0] = 0.0
        cnt[...] = jnp.zeros_like(cnt)

    @pl.when(i >= 2 * SA)
    def _phase_main():
        emb = emb_ref[...]
        esq = jnp.sum(emb * emb, axis=1, keepdims=True)
        kio = lax.broadcasted_iota(jnp.int32, (K, HW), 0)
        for j in range(BB):
            zn = z_ref[j] / std_s[...]
            mm = lax.dot_general(emb, zn, (((1,), (0,)), ((), ())),
                                 preferred_element_type=jnp.float32)
            zsq = jnp.sum(zn * zn, axis=0, keepdims=True)
            dist = (esq - 2.0 * mm) + zsq
            md = jnp.min(dist, axis=0)
            idx = jnp.min(jnp.where(dist == md[None, :], kio, K), axis=0)
            err_s[0, 0] += jnp.sum(md)
            oh = (kio == idx[None, :]).astype(jnp.float32)
            cnt[...] += jnp.sum(oh, axis=1, keepdims=True)
            idx_ref[j, 0, :] = idx

    @pl.when(i == 3 * SA - 1)
    def _finalize():
        loss_ref[0, 0] = 1.25 * err_s[0, 0] / NELEM
        p = cnt[...] / N
        plogp = p * jnp.log(jnp.maximum(p, 1e-10))
        perp_ref[0, 0] = jnp.exp(-jnp.sum(plogp))


def _sc_body(embt_hbm, idx_hbm, out_hbm, embt_v, idx_v, obuf):
    cid = lax.axis_index("c")
    sid = lax.axis_index("s")
    wid = sid * 2 + cid
    pltpu.sync_copy(embt_hbm, embt_v)

    def per_batch(bl, carry):
        b = wid * 2 + bl
        pltpu.sync_copy(idx_hbm.at[b], idx_v)

        @plsc.parallel_loop(0, HW // 16, unroll=2)
        def per_group(g):
            ids = idx_v[pl.ds(g * 16, 16)]
            for c in range(C):
                obuf[c, pl.ds(g * 16, 16)] = plsc.load_gather(embt_v, [ids + c * K])

        pltpu.sync_copy(obuf, out_hbm.at[b])
        return carry

    lax.fori_loop(0, 2, per_batch, 0)


def _make_sc_gather():
    mesh = plsc.VectorSubcoreMesh(core_axis_name="c", subcore_axis_name="s")
    return functools.partial(
        pl.kernel,
        mesh=mesh,
        compiler_params=pltpu.CompilerParams(needs_layout_passes=False),
        out_type=jax.ShapeDtypeStruct((B, C, HW), jnp.float32),
        scratch_types=[
            pltpu.VMEM((C * K,), jnp.float32),
            pltpu.VMEM((HW,), jnp.int32),
            pltpu.VMEM((C, HW), jnp.float32),
        ],
    )(_sc_body)


def kernel(z_e, emb_w):
    z3 = z_e.reshape(B, C, HW)
    idx3, loss, perp, embt = pl.pallas_call(
        _tc_body,
        grid=(3 * SA,),
        in_specs=[
            pl.BlockSpec((BB, C, HW), lambda i: (i % SA, 0, 0)),
            pl.BlockSpec((K, C), lambda i: (0, 0)),
        ],
        out_specs=[
            pl.BlockSpec((BB, 1, HW), lambda i: (jnp.maximum(i - 2 * SA, 0), 0, 0)),
            pl.BlockSpec(memory_space=pltpu.SMEM),
            pl.BlockSpec(memory_space=pltpu.SMEM),
            pl.BlockSpec((C, K), lambda i: (0, 0)),
        ],
        out_shape=[
            jax.ShapeDtypeStruct((B, 1, HW), jnp.int32),
            jax.ShapeDtypeStruct((1, 1), jnp.float32),
            jax.ShapeDtypeStruct((1, 1), jnp.float32),
            jax.ShapeDtypeStruct((C, K), jnp.float32),
        ],
        scratch_shapes=[
            pltpu.VMEM((C, HW), jnp.float32),
            pltpu.VMEM((C, 1), jnp.float32),
            pltpu.VMEM((C, 1), jnp.float32),
            pltpu.SMEM((1, 1), jnp.float32),
            pltpu.VMEM((K, 1), jnp.float32),
        ],
    )(z3, emb_w)
    idx2 = idx3.reshape(B, HW)
    zq = _make_sc_gather()(embt.reshape(-1), idx2)
    z_q_st = zq.reshape(z_e.shape)
    indices = idx2.reshape(B, 32, 32)
    return (z_q_st, loss[0, 0], perp[0, 0], indices)


# speedup vs baseline: 1.9577x; 1.0603x over previous
import functools

import jax
import jax.numpy as jnp
from jax import lax
from jax.experimental import pallas as pl
from jax.experimental.pallas import tpu as pltpu
from jax.experimental.pallas import tpu_sc as plsc

B = 64
C = 64
HW = 1024
K = 512
N = B * HW
NELEM = N * C
BB = 8
SA = B // BB


def _tc_body(z_ref, emb_ref, idx_ref, loss_ref, perp_ref, embt_ref,
             acc, mean_s, std_s, err_s, cnt):
    i = pl.program_id(0)

    @pl.when(i == 0)
    def _init():
        acc[...] = jnp.zeros_like(acc)

    @pl.when(i < SA)
    def _phase_sum():
        for j in range(BB):
            acc[...] += z_ref[j]

    @pl.when(i == SA)
    def _fin_mean():
        mean_s[...] = jnp.sum(acc[...], axis=1, keepdims=True) / N
        acc[...] = jnp.zeros_like(acc)

    @pl.when(jnp.logical_and(i >= SA, i < 2 * SA))
    def _phase_sq():
        for j in range(BB):
            d = z_ref[j] - mean_s[...]
            acc[...] += d * d

    @pl.when(i == 2 * SA)
    def _fin_std():
        var = jnp.sum(acc[...], axis=1, keepdims=True) / (N - 1)
        std_s[...] = jnp.maximum(jnp.sqrt(var), 1e-5)
        embt_ref[...] = emb_ref[...].T
        err_s[0, 0] = 0.0
        cnt[...] = jnp.zeros_like(cnt)

    @pl.when(i >= 2 * SA)
    def _phase_main():
        emb = emb_ref[...]
        esq = jnp.sum(emb * emb, axis=1, keepdims=True)
        kio = lax.broadcasted_iota(jnp.int32, (K, HW), 0)
        for j in range(BB):
            zn = z_ref[j] / std_s[...]
            mm = lax.dot_general(emb, zn, (((1,), (0,)), ((), ())),
                                 preferred_element_type=jnp.float32)
            zsq = jnp.sum(zn * zn, axis=0, keepdims=True)
            dist = (esq - 2.0 * mm) + zsq
            md = jnp.min(dist, axis=0)
            idx = jnp.min(jnp.where(dist == md[None, :], kio, K), axis=0)
            err_s[0, 0] += jnp.sum(md)
            oh = (kio == idx[None, :]).astype(jnp.float32)
            cnt[...] += jnp.sum(oh, axis=1, keepdims=True)
            idx_ref[j, 0, :] = idx

    @pl.when(i == 3 * SA - 1)
    def _finalize():
        loss_ref[0, 0] = 1.25 * err_s[0, 0] / NELEM
        p = cnt[...] / N
        plogp = p * jnp.log(jnp.maximum(p, 1e-10))
        perp_ref[0, 0] = jnp.exp(-jnp.sum(plogp))


def _sc_body(embt_hbm, idx_hbm, out_hbm, embt_v, idx_v, obuf):
    cid = lax.axis_index("c")
    sid = lax.axis_index("s")
    wid = sid * 2 + cid
    pltpu.sync_copy(embt_hbm, embt_v)

    def per_batch(bl, carry):
        b = wid * 2 + bl
        pltpu.sync_copy(idx_hbm.at[b], idx_v)

        @plsc.parallel_loop(0, HW // 16, unroll=2)
        def per_group(g):
            ids = idx_v[pl.ds(g * 16, 16)]
            for c in range(C):
                obuf[c, pl.ds(g * 16, 16)] = plsc.load_gather(embt_v, [ids + c * K])

        pltpu.sync_copy(obuf, out_hbm.at[b])
        return carry

    lax.fori_loop(0, 2, per_batch, 0)


def _make_sc_gather():
    mesh = plsc.VectorSubcoreMesh(core_axis_name="c", subcore_axis_name="s")
    return functools.partial(
        pl.kernel,
        mesh=mesh,
        compiler_params=pltpu.CompilerParams(needs_layout_passes=False),
        out_type=jax.ShapeDtypeStruct((B, C, HW), jnp.float32),
        scratch_types=[
            pltpu.VMEM((C * K,), jnp.float32),
            pltpu.VMEM((HW,), jnp.int32),
            pltpu.VMEM((C, HW), jnp.float32),
        ],
    )(_sc_body)


def kernel(z_e, emb_w):
    z3 = z_e.reshape(B, C, HW)
    idx3, loss, perp, embt = pl.pallas_call(
        _tc_body,
        grid=(3 * SA,),
        in_specs=[
            pl.BlockSpec((BB, C, HW), lambda i: (i % SA, 0, 0)),
            pl.BlockSpec((K, C), lambda i: (0, 0)),
        ],
        out_specs=[
            pl.BlockSpec((BB, 1, HW), lambda i: (jnp.maximum(i - 2 * SA, 0), 0, 0)),
            pl.BlockSpec(memory_space=pltpu.SMEM),
            pl.BlockSpec(memory_space=pltpu.SMEM),
            pl.BlockSpec((C, K), lambda i: (0, 0)),
        ],
        out_shape=[
            jax.ShapeDtypeStruct((B, 1, HW), jnp.int32),
            jax.ShapeDtypeStruct((1, 1), jnp.float32),
            jax.ShapeDtypeStruct((1, 1), jnp.float32),
            jax.ShapeDtypeStruct((C, K), jnp.float32),
        ],
        scratch_shapes=[
            pltpu.VMEM((C, HW), jnp.float32),
            pltpu.VMEM((C, 1), jnp.float32),
            pltpu.VMEM((C, 1), jnp.float32),
            pltpu.SMEM((1, 1), jnp.float32),
            pltpu.VMEM((K, 1), jnp.float32),
        ],
    )(z3, emb_w)
    idx2 = idx3.reshape(B, HW)
    zq = _make_sc_gather()(embt.reshape(-1), idx2)
    z_q_st = zq.reshape(z_e.shape)
    indices = idx2.reshape(B, 32, 32)
    return (z_q_st, loss[0, 0], perp[0, 0], indices)


# speedup vs baseline: 2.1472x; 1.0968x over previous
import functools

import jax
import jax.numpy as jnp
from jax import lax
from jax.experimental import pallas as pl
from jax.experimental.pallas import tpu as pltpu
from jax.experimental.pallas import tpu_sc as plsc

B = 64
C = 64
HW = 1024
K = 512
N = B * HW
NELEM = N * C
BB = 8
SA = B // BB
NW = 32


def _tc_body(z_ref, emb_ref, idx_ref, loss_ref, embt_ref,
             acc, mean_s, std_s, n2emb_s, esq_s, err_s):
    i = pl.program_id(0)

    @pl.when(i == 0)
    def _init():
        acc[...] = jnp.zeros_like(acc)

    @pl.when(i < SA)
    def _phase_sum():
        for j in range(BB):
            acc[...] += z_ref[j]

    @pl.when(i == SA)
    def _fin_mean():
        mean_s[...] = jnp.sum(acc[...], axis=1, keepdims=True) / N
        acc[...] = jnp.zeros_like(acc)

    @pl.when(jnp.logical_and(i >= SA, i < 2 * SA))
    def _phase_sq():
        for j in range(BB):
            d = z_ref[j] - mean_s[...]
            acc[...] += d * d

    @pl.when(i == 2 * SA)
    def _fin_std():
        var = jnp.sum(acc[...], axis=1, keepdims=True) / (N - 1)
        std_s[...] = jnp.maximum(jnp.sqrt(var), 1e-5)
        emb = emb_ref[...]
        embt_ref[...] = emb.T
        n2emb_s[...] = emb * -2.0
        esq_s[...] = jnp.sum(emb * emb, axis=1, keepdims=True)
        err_s[0, 0] = 0.0

    @pl.when(i >= 2 * SA)
    def _phase_main():
        kio = lax.broadcasted_iota(jnp.int32, (K, HW), 0)
        for j in range(BB):
            zn = z_ref[j] / std_s[...]
            mm2 = lax.dot_general(n2emb_s[...], zn, (((1,), (0,)), ((), ())),
                                  preferred_element_type=jnp.float32)
            zsq = jnp.sum(zn * zn, axis=0, keepdims=True)
            dist = (zsq + mm2) + esq_s[...]
            md = jnp.min(dist, axis=0)
            idx = jnp.min(jnp.where(dist == md[None, :], kio, K), axis=0)
            err_s[0, 0] += jnp.sum(md)
            idx_ref[j, 0, :] = idx

    @pl.when(i == 3 * SA - 1)
    def _finalize():
        loss_ref[0, 0] = 1.25 * err_s[0, 0] / NELEM


def _sc_body(embt_hbm, idx_hbm, out_hbm, cnt_hbm, embt_v, idx_v, obuf, cnt_v):
    cid = lax.axis_index("c")
    sid = lax.axis_index("s")
    wid = sid * 2 + cid
    pltpu.sync_copy(embt_hbm, embt_v)

    def zero_cnt(g, carry):
        cnt_v[pl.ds(g * 16, 16)] = jnp.zeros((16,), jnp.float32)
        return carry

    lax.fori_loop(0, K // 16, zero_cnt, 0)

    def per_batch(bl, carry):
        b = wid * 2 + bl
        pltpu.sync_copy(idx_hbm.at[b], idx_v)

        @plsc.parallel_loop(0, HW // 16, unroll=2)
        def per_group(g):
            ids = idx_v[pl.ds(g * 16, 16)]
            for c in range(C):
                obuf[c, pl.ds(g * 16, 16)] = plsc.load_gather(embt_v, [ids + c * K])

        def hist(g, carry2):
            ids = idx_v[pl.ds(g * 16, 16)]
            plsc.addupdate_scatter(cnt_v, [ids], jnp.ones((16,), jnp.float32))
            return carry2

        lax.fori_loop(0, HW // 16, hist, 0)
        pltpu.sync_copy(obuf, out_hbm.at[b])
        return carry

    lax.fori_loop(0, 2, per_batch, 0)
    pltpu.sync_copy(cnt_v, cnt_hbm.at[wid])


def _make_sc_gather():
    mesh = plsc.VectorSubcoreMesh(core_axis_name="c", subcore_axis_name="s")
    return functools.partial(
        pl.kernel,
        mesh=mesh,
        compiler_params=pltpu.CompilerParams(needs_layout_passes=False),
        out_type=[
            jax.ShapeDtypeStruct((B, C, HW), jnp.float32),
            jax.ShapeDtypeStruct((NW, K), jnp.float32),
        ],
        scratch_types=[
            pltpu.VMEM((C * K,), jnp.float32),
            pltpu.VMEM((HW,), jnp.int32),
            pltpu.VMEM((C, HW), jnp.float32),
            pltpu.VMEM((K,), jnp.float32),
        ],
    )(_sc_body)


def _perp_body(cnt_ref, perp_ref):
    p = jnp.sum(cnt_ref[...], axis=0, keepdims=True) / N
    plogp = p * jnp.log(jnp.maximum(p, 1e-10))
    perp_ref[0, 0] = jnp.exp(-jnp.sum(plogp))


def kernel(z_e, emb_w):
    z3 = z_e.reshape(B, C, HW)
    idx3, loss, embt = pl.pallas_call(
        _tc_body,
        grid=(3 * SA,),
        in_specs=[
            pl.BlockSpec((BB, C, HW), lambda i: (i % SA, 0, 0)),
            pl.BlockSpec((K, C), lambda i: (0, 0)),
        ],
        out_specs=[
            pl.BlockSpec((BB, 1, HW), lambda i: (jnp.maximum(i - 2 * SA, 0), 0, 0)),
            pl.BlockSpec(memory_space=pltpu.SMEM),
            pl.BlockSpec((C, K), lambda i: (0, 0)),
        ],
        out_shape=[
            jax.ShapeDtypeStruct((B, 1, HW), jnp.int32),
            jax.ShapeDtypeStruct((1, 1), jnp.float32),
            jax.ShapeDtypeStruct((C, K), jnp.float32),
        ],
        scratch_shapes=[
            pltpu.VMEM((C, HW), jnp.float32),
            pltpu.VMEM((C, 1), jnp.float32),
            pltpu.VMEM((C, 1), jnp.float32),
            pltpu.VMEM((K, C), jnp.float32),
            pltpu.VMEM((K, 1), jnp.float32),
            pltpu.SMEM((1, 1), jnp.float32),
        ],
    )(z3, emb_w)
    idx2 = idx3.reshape(B, HW)
    zq, cnts = _make_sc_gather()(embt.reshape(-1), idx2)
    perp = pl.pallas_call(
        _perp_body,
        out_specs=pl.BlockSpec(memory_space=pltpu.SMEM),
        out_shape=jax.ShapeDtypeStruct((1, 1), jnp.float32),
    )(cnts)
    z_q_st = zq.reshape(z_e.shape)
    indices = idx2.reshape(B, 32, 32)
    return (z_q_st, loss[0, 0], perp[0, 0], indices)


# speedup vs baseline: 2.2796x; 1.0616x over previous
import functools

import jax
import jax.numpy as jnp
from jax import lax
from jax.experimental import pallas as pl
from jax.experimental.pallas import tpu as pltpu
from jax.experimental.pallas import tpu_sc as plsc

B = 64
C = 64
HW = 1024
K = 512
N = B * HW
NELEM = N * C
BB = 16
SA = B // BB
NW = 32


def _tc_body(z_ref, emb_ref, idx_ref, loss_ref, embt_ref,
             acc, mean_s, std_s, n2emb_s, esq_s, err_s):
    i = pl.program_id(0)

    @pl.when(i == 0)
    def _init():
        acc[...] = jnp.zeros_like(acc)

    @pl.when(i < SA)
    def _phase_sum():
        for j in range(BB):
            acc[...] += z_ref[(i % SA) * BB + j]

    @pl.when(i == SA)
    def _fin_mean():
        mean_s[...] = jnp.sum(acc[...], axis=1, keepdims=True) / N
        acc[...] = jnp.zeros_like(acc)

    @pl.when(jnp.logical_and(i >= SA, i < 2 * SA))
    def _phase_sq():
        for j in range(BB):
            d = z_ref[(i % SA) * BB + j] - mean_s[...]
            acc[...] += d * d

    @pl.when(i == 2 * SA)
    def _fin_std():
        var = jnp.sum(acc[...], axis=1, keepdims=True) / (N - 1)
        std_s[...] = jnp.maximum(jnp.sqrt(var), 1e-5)
        emb = emb_ref[...]
        embt_ref[...] = emb.T
        n2emb_s[...] = emb * -2.0
        esq_s[...] = jnp.sum(emb * emb, axis=1, keepdims=True)
        err_s[0, 0] = 0.0

    @pl.when(i >= 2 * SA)
    def _phase_main():
        kio = lax.broadcasted_iota(jnp.int32, (K, HW), 0)
        for j in range(BB):
            zn = z_ref[(i % SA) * BB + j] / std_s[...]
            mm2 = lax.dot_general(n2emb_s[...], zn, (((1,), (0,)), ((), ())),
                                  preferred_element_type=jnp.float32)
            zsq = jnp.sum(zn * zn, axis=0, keepdims=True)
            dist = (zsq + mm2) + esq_s[...]
            md = jnp.min(dist, axis=0)
            idx = jnp.min(jnp.where(dist == md[None, :], kio, K), axis=0)
            err_s[0, 0] += jnp.sum(md)
            idx_ref[j, 0, :] = idx

    @pl.when(i == 3 * SA - 1)
    def _finalize():
        loss_ref[0, 0] = 1.25 * err_s[0, 0] / NELEM


def _sc_body(embt_hbm, idx_hbm, out_hbm, cnt_hbm, embt_v, idx_v, obuf, cnt_v):
    cid = lax.axis_index("c")
    sid = lax.axis_index("s")
    wid = sid * 2 + cid
    pltpu.sync_copy(embt_hbm, embt_v)

    def zero_cnt(g, carry):
        cnt_v[pl.ds(g * 16, 16)] = jnp.zeros((16,), jnp.float32)
        return carry

    lax.fori_loop(0, K // 16, zero_cnt, 0)

    def per_batch(bl, carry):
        b = wid * 2 + bl
        pltpu.sync_copy(idx_hbm.at[b], idx_v)

        @plsc.parallel_loop(0, HW // 16, unroll=2)
        def per_group(g):
            ids = idx_v[pl.ds(g * 16, 16)]
            for c in range(C):
                obuf[c, pl.ds(g * 16, 16)] = plsc.load_gather(embt_v, [ids + c * K])

        def hist(g, carry2):
            ids = idx_v[pl.ds(g * 16, 16)]
            plsc.addupdate_scatter(cnt_v, [ids], jnp.ones((16,), jnp.float32))
            return carry2

        lax.fori_loop(0, HW // 16, hist, 0)
        pltpu.sync_copy(obuf, out_hbm.at[b])
        return carry

    lax.fori_loop(0, 2, per_batch, 0)
    pltpu.sync_copy(cnt_v, cnt_hbm.at[wid])


def _make_sc_gather():
    mesh = plsc.VectorSubcoreMesh(core_axis_name="c", subcore_axis_name="s")
    return functools.partial(
        pl.kernel,
        mesh=mesh,
        compiler_params=pltpu.CompilerParams(needs_layout_passes=False),
        out_type=[
            jax.ShapeDtypeStruct((B, C, HW), jnp.float32),
            jax.ShapeDtypeStruct((NW, K), jnp.float32),
        ],
        scratch_types=[
            pltpu.VMEM((C * K,), jnp.float32),
            pltpu.VMEM((HW,), jnp.int32),
            pltpu.VMEM((C, HW), jnp.float32),
            pltpu.VMEM((K,), jnp.float32),
        ],
    )(_sc_body)


def _perp_body(cnt_ref, perp_ref):
    p = jnp.sum(cnt_ref[...], axis=0, keepdims=True) / N
    plogp = p * jnp.log(jnp.maximum(p, 1e-10))
    perp_ref[0, 0] = jnp.exp(-jnp.sum(plogp))


def kernel(z_e, emb_w):
    z3 = z_e.reshape(B, C, HW)
    idx3, loss, embt = pl.pallas_call(
        _tc_body,
        grid=(3 * SA,),
        in_specs=[
            pl.BlockSpec((B, C, HW), lambda i: (0, 0, 0)),
            pl.BlockSpec((K, C), lambda i: (0, 0)),
        ],
        out_specs=[
            pl.BlockSpec((BB, 1, HW), lambda i: (jnp.maximum(i - 2 * SA, 0), 0, 0)),
            pl.BlockSpec(memory_space=pltpu.SMEM),
            pl.BlockSpec((C, K), lambda i: (0, 0)),
        ],
        out_shape=[
            jax.ShapeDtypeStruct((B, 1, HW), jnp.int32),
            jax.ShapeDtypeStruct((1, 1), jnp.float32),
            jax.ShapeDtypeStruct((C, K), jnp.float32),
        ],
        scratch_shapes=[
            pltpu.VMEM((C, HW), jnp.float32),
            pltpu.VMEM((C, 1), jnp.float32),
            pltpu.VMEM((C, 1), jnp.float32),
            pltpu.VMEM((K, C), jnp.float32),
            pltpu.VMEM((K, 1), jnp.float32),
            pltpu.SMEM((1, 1), jnp.float32),
        ],
    )(z3, emb_w)
    idx2 = idx3.reshape(B, HW)
    zq, cnts = _make_sc_gather()(embt.reshape(-1), idx2)
    perp = pl.pallas_call(
        _perp_body,
        out_specs=pl.BlockSpec(memory_space=pltpu.SMEM),
        out_shape=jax.ShapeDtypeStruct((1, 1), jnp.float32),
    )(cnts)
    z_q_st = zq.reshape(z_e.shape)
    indices = idx2.reshape(B, 32, 32)
    return (z_q_st, loss[0, 0], perp[0, 0], indices)
